# Initial kernel scaffold; baseline (speedup 1.0000x reference)
import jax
import jax.numpy as jnp
from jax.experimental import pallas as pl


def kernel(x, true_target, neg_targets, w_t):
    raise NotImplementedError("write your pallas kernel here")


# speedup vs baseline: 2.3990x; 2.3990x over previous
import functools

import jax
import jax.numpy as jnp
from jax import lax
from jax.experimental import pallas as pl
from jax.experimental.pallas import tpu as pltpu
from jax.experimental.pallas import tpu_sc as plsc

_CP = 128


def _sc_gather(w_t, tt2d, nt2d):
    D = w_t.shape[1]
    B = tt2d.shape[0] * _CP
    S = nt2d.shape[0] * _CP
    info = plsc.get_sparse_core_info()
    NC, NS = info.num_cores, info.num_subcores
    NW = NC * NS
    pos_chunks = B // _CP // NW
    neg_chunks = S // _CP // NW
    mesh = plsc.VectorSubcoreMesh(core_axis_name="c", subcore_axis_name="s")

    @functools.partial(
        pl.kernel,
        mesh=mesh,
        out_type=(
            jax.ShapeDtypeStruct((B, D), jnp.float32),
            jax.ShapeDtypeStruct((S, D), jnp.float32),
        ),
        scratch_types=[
            pltpu.VMEM((pos_chunks, _CP), jnp.int32),
            pltpu.VMEM((neg_chunks, _CP), jnp.int32),
            pltpu.VMEM((pos_chunks * _CP, D), jnp.float32),
            pltpu.VMEM((neg_chunks * _CP, D), jnp.float32),
            pltpu.SemaphoreType.DMA,
        ],
    )
    def k(w_hbm, tt_hbm, nt_hbm, pos_out, neg_out, tt_v, nt_v, posw_v,
          negw_v, sem):
        wid = lax.axis_index("s") * NC + lax.axis_index("c")
        pltpu.sync_copy(tt_hbm.at[pl.ds(wid * pos_chunks, pos_chunks)], tt_v)
        pltpu.sync_copy(nt_hbm.at[pl.ds(wid * neg_chunks, neg_chunks)], nt_v)
        descs = []
        for j in range(pos_chunks):
            descs.append(pltpu.async_copy(
                w_hbm.at[tt_v.at[j]], posw_v.at[pl.ds(j * _CP, _CP)], sem))
        for j in range(neg_chunks):
            descs.append(pltpu.async_copy(
                w_hbm.at[nt_v.at[j]], negw_v.at[pl.ds(j * _CP, _CP)], sem))
        for d in descs:
            d.wait()
        pltpu.sync_copy(
            posw_v, pos_out.at[pl.ds(wid * pos_chunks * _CP, pos_chunks * _CP)])
        pltpu.sync_copy(
            negw_v, neg_out.at[pl.ds(wid * neg_chunks * _CP, neg_chunks * _CP)])

    return k(w_t, tt2d, nt2d)


def _tc_compute(x, pos_w, neg_w, bm=512, bn=2048):
    B, D = x.shape
    S = neg_w.shape[0]

    def body(x_ref, pw_ref, nw_ref, neg_ref, pos_ref):
        neg_ref[...] = lax.dot_general(
            x_ref[...], nw_ref[...], (((1,), (1,)), ((), ())),
            preferred_element_type=jnp.float32)

        @pl.when(pl.program_id(1) == 0)
        def _():
            pos_ref[...] = jnp.sum(x_ref[...] * pw_ref[...], axis=1)

    neg_logits, pos_logits = pl.pallas_call(
        body,
        grid=(B // bm, S // bn),
        in_specs=[
            pl.BlockSpec((bm, D), lambda i, j: (i, 0)),
            pl.BlockSpec((bm, D), lambda i, j: (i, 0)),
            pl.BlockSpec((bn, D), lambda i, j: (j, 0)),
        ],
        out_specs=[
            pl.BlockSpec((bm, bn), lambda i, j: (i, j)),
            pl.BlockSpec((bm,), lambda i, j: (i,)),
        ],
        out_shape=[
            jax.ShapeDtypeStruct((B, S), jnp.float32),
            jax.ShapeDtypeStruct((B,), jnp.float32),
        ],
        compiler_params=pltpu.CompilerParams(
            dimension_semantics=("parallel", "arbitrary")),
    )(x, pos_w, neg_w)
    return pos_logits, neg_logits


def kernel(x, true_target, neg_targets, w_t):
    B, _ = x.shape
    S = neg_targets.shape[0]
    tt2d = true_target.astype(jnp.int32).reshape(B // _CP, _CP)
    nt2d = neg_targets.astype(jnp.int32).reshape(S // _CP, _CP)
    pos_w, neg_w = _sc_gather(w_t, tt2d, nt2d)
    return _tc_compute(x, pos_w, neg_w)


# speedup vs baseline: 2.7015x; 1.1261x over previous
import functools

import jax
import jax.numpy as jnp
from jax import lax
from jax.experimental import pallas as pl
from jax.experimental.pallas import tpu as pltpu
from jax.experimental.pallas import tpu_sc as plsc

_CP = 128


def _sc_gather(w_t, tt2d, nt2d):
    D = w_t.shape[1]
    B = tt2d.shape[0] * _CP
    S = nt2d.shape[0] * _CP
    info = plsc.get_sparse_core_info()
    NC, NS = info.num_cores, info.num_subcores
    NW = NC * NS
    pos_chunks = B // _CP // NW
    neg_chunks = S // _CP // NW
    mesh = plsc.VectorSubcoreMesh(core_axis_name="c", subcore_axis_name="s")

    @functools.partial(
        pl.kernel,
        mesh=mesh,
        out_type=(
            jax.ShapeDtypeStruct((B, D), jnp.float32),
            jax.ShapeDtypeStruct((S, D), jnp.float32),
        ),
        scratch_types=[
            pltpu.VMEM((pos_chunks, _CP), jnp.int32),
            pltpu.VMEM((neg_chunks, _CP), jnp.int32),
            pltpu.VMEM((pos_chunks * _CP, D), jnp.float32),
            pltpu.VMEM((neg_chunks * _CP, D), jnp.float32),
            pltpu.SemaphoreType.DMA,
        ],
    )
    def k(w_hbm, tt_hbm, nt_hbm, pos_out, neg_out, tt_v, nt_v, posw_v,
          negw_v, sem):
        wid = lax.axis_index("s") * NC + lax.axis_index("c")
        pltpu.sync_copy(tt_hbm.at[pl.ds(wid * pos_chunks, pos_chunks)], tt_v)
        pltpu.sync_copy(nt_hbm.at[pl.ds(wid * neg_chunks, neg_chunks)], nt_v)
        descs = []
        for j in range(pos_chunks):
            descs.append(pltpu.async_copy(
                w_hbm.at[tt_v.at[j]], posw_v.at[pl.ds(j * _CP, _CP)], sem))
        for j in range(neg_chunks):
            descs.append(pltpu.async_copy(
                w_hbm.at[nt_v.at[j]], negw_v.at[pl.ds(j * _CP, _CP)], sem))
        for d in descs:
            d.wait()
        pltpu.sync_copy(
            posw_v, pos_out.at[pl.ds(wid * pos_chunks * _CP, pos_chunks * _CP)])
        pltpu.sync_copy(
            negw_v, neg_out.at[pl.ds(wid * neg_chunks * _CP, neg_chunks * _CP)])

    return k(w_t, tt2d, nt2d)


def _tc_compute(x, pos_w, neg_w, bm=512, bn=4096):
    B, D = x.shape
    S = neg_w.shape[0]

    def body(x_ref, pw_ref, nw_ref, neg_ref, pos_ref):
        neg_ref[...] = lax.dot_general(
            x_ref[...], nw_ref[...], (((1,), (1,)), ((), ())),
            preferred_element_type=jnp.float32)

        @pl.when(pl.program_id(1) == 0)
        def _():
            pos_ref[...] = jnp.sum(x_ref[...] * pw_ref[...], axis=1)

    neg_logits, pos_logits = pl.pallas_call(
        body,
        grid=(B // bm, S // bn),
        in_specs=[
            pl.BlockSpec((bm, D), lambda i, j: (i, 0)),
            pl.BlockSpec((bm, D), lambda i, j: (i, 0)),
            pl.BlockSpec((bn, D), lambda i, j: (j, 0)),
        ],
        out_specs=[
            pl.BlockSpec((bm, bn), lambda i, j: (i, j)),
            pl.BlockSpec((bm,), lambda i, j: (i,)),
        ],
        out_shape=[
            jax.ShapeDtypeStruct((B, S), jnp.float32),
            jax.ShapeDtypeStruct((B,), jnp.float32),
        ],
        compiler_params=pltpu.CompilerParams(
            dimension_semantics=("parallel", "arbitrary")),
    )(x, pos_w, neg_w)
    return pos_logits, neg_logits


def kernel(x, true_target, neg_targets, w_t):
    B, _ = x.shape
    S = neg_targets.shape[0]
    tt2d = true_target.astype(jnp.int32).reshape(B // _CP, _CP)
    nt2d = neg_targets.astype(jnp.int32).reshape(S // _CP, _CP)
    pos_w, neg_w = _sc_gather(w_t, tt2d, nt2d)
    return _tc_compute(x, pos_w, neg_w)


# speedup vs baseline: 3.3768x; 1.2500x over previous
import functools

import jax
import jax.numpy as jnp
from jax import lax
from jax.experimental import pallas as pl
from jax.experimental.pallas import tpu as pltpu
from jax.experimental.pallas import tpu_sc as plsc

_CP = 128


def _sc_gather(w_t, tt2d, nt2d):
    D = w_t.shape[1]
    B = tt2d.shape[0] * _CP
    S = nt2d.shape[0] * _CP
    info = plsc.get_sparse_core_info()
    NC, NS = info.num_cores, info.num_subcores
    NW = NC * NS
    pos_chunks = B // _CP // NW
    neg_chunks = S // _CP // NW
    mesh = plsc.VectorSubcoreMesh(core_axis_name="c", subcore_axis_name="s")

    @functools.partial(
        pl.kernel,
        mesh=mesh,
        out_type=(
            jax.ShapeDtypeStruct((B, D), jnp.float32),
            jax.ShapeDtypeStruct((S, D), jnp.float32),
        ),
        scratch_types=[
            pltpu.VMEM((pos_chunks, _CP), jnp.int32),
            pltpu.VMEM((neg_chunks, _CP), jnp.int32),
            pltpu.VMEM((pos_chunks * _CP, D), jnp.float32),
            pltpu.VMEM((neg_chunks * _CP, D), jnp.float32),
            pltpu.SemaphoreType.DMA,
        ],
    )
    def k(w_hbm, tt_hbm, nt_hbm, pos_out, neg_out, tt_v, nt_v, posw_v,
          negw_v, sem):
        wid = lax.axis_index("s") * NC + lax.axis_index("c")
        pltpu.sync_copy(tt_hbm.at[pl.ds(wid * pos_chunks, pos_chunks)], tt_v)
        pltpu.sync_copy(nt_hbm.at[pl.ds(wid * neg_chunks, neg_chunks)], nt_v)
        descs = []
        for j in range(pos_chunks):
            descs.append(pltpu.async_copy(
                w_hbm.at[tt_v.at[j]], posw_v.at[pl.ds(j * _CP, _CP)], sem))
        for j in range(neg_chunks):
            descs.append(pltpu.async_copy(
                w_hbm.at[nt_v.at[j]], negw_v.at[pl.ds(j * _CP, _CP)], sem))
        for d in descs:
            d.wait()
        pltpu.sync_copy(
            posw_v, pos_out.at[pl.ds(wid * pos_chunks * _CP, pos_chunks * _CP)])
        pltpu.sync_copy(
            negw_v, neg_out.at[pl.ds(wid * neg_chunks * _CP, neg_chunks * _CP)])

    return k(w_t, tt2d, nt2d)


def _tc_compute(x, pos_w, neg_w, bm=512, bn=8192):
    B, D = x.shape
    S = neg_w.shape[0]

    def body(x_ref, pw_ref, nw_ref, neg_ref, pos_ref):
        neg_ref[...] = lax.dot_general(
            x_ref[...], nw_ref[...], (((1,), (1,)), ((), ())),
            preferred_element_type=jnp.float32)

        @pl.when(pl.program_id(1) == 0)
        def _():
            pos_ref[...] = jnp.sum(x_ref[...] * pw_ref[...], axis=1)

    neg_logits, pos_logits = pl.pallas_call(
        body,
        grid=(B // bm, S // bn),
        in_specs=[
            pl.BlockSpec((bm, D), lambda i, j: (i, 0)),
            pl.BlockSpec((bm, D), lambda i, j: (i, 0)),
            pl.BlockSpec((bn, D), lambda i, j: (j, 0)),
        ],
        out_specs=[
            pl.BlockSpec((bm, bn), lambda i, j: (i, j)),
            pl.BlockSpec((bm,), lambda i, j: (i,)),
        ],
        out_shape=[
            jax.ShapeDtypeStruct((B, S), jnp.float32),
            jax.ShapeDtypeStruct((B,), jnp.float32),
        ],
        compiler_params=pltpu.CompilerParams(
            dimension_semantics=("parallel", "arbitrary")),
    )(x, pos_w, neg_w)
    return pos_logits, neg_logits


def kernel(x, true_target, neg_targets, w_t):
    B, _ = x.shape
    S = neg_targets.shape[0]
    tt2d = true_target.astype(jnp.int32).reshape(B // _CP, _CP)
    nt2d = neg_targets.astype(jnp.int32).reshape(S // _CP, _CP)
    pos_w, neg_w = _sc_gather(w_t, tt2d, nt2d)
    return _tc_compute(x, pos_w, neg_w)
